# baseline (speedup 1.0000x reference)
import functools

import jax
import jax.numpy as jnp
from jax import lax
from jax.experimental import pallas as pl
from jax.experimental.pallas import tpu as pltpu
from jax.experimental.pallas import tpu_sc as plsc

VOCAB = 49408
MAX_POS = 77
EMBED = 512
BATCH = 4096
SEQ = 77
SEQ_PAD = 80

NUM_CORES = 2
NUM_SUBCORES = 16
NUM_WORKERS = NUM_CORES * NUM_SUBCORES
BPW = BATCH // NUM_WORKERS
LANES = 16


def _impl(ids_hbm, pos_ids_hbm, tok_hbm, pos_hbm, out_hbm,
          pos_idx, pos_rows, idx_buf, rows, gsem):
    wid = lax.axis_index("s") * NUM_CORES + lax.axis_index("c")
    b0 = wid * BPW

    pltpu.sync_copy(pos_ids_hbm.at[0], pos_idx)
    pltpu.async_copy(pos_hbm.at[pos_idx], pos_rows, gsem).wait()

    def body(g, carry):
        b = b0 + g
        pltpu.sync_copy(ids_hbm.at[b], idx_buf)
        pltpu.async_copy(tok_hbm.at[idx_buf], rows, gsem).wait()

        def add_row(r, c):
            for j in range(EMBED // LANES):
                sl = pl.ds(j * LANES, LANES)
                plsc.addupdate(rows.at[r, sl], pos_rows[r, sl])
            return c

        lax.fori_loop(0, SEQ, add_row, 0)
        pltpu.sync_copy(rows.at[pl.ds(0, SEQ)], out_hbm.at[b])
        return carry

    lax.fori_loop(0, BPW, body, 0)


@jax.jit
def kernel(input_ids, position_ids, token_table, position_table):
    ids_pad = jnp.pad(input_ids.astype(jnp.int32), ((0, 0), (0, SEQ_PAD - SEQ)))
    pos_ids_pad = jnp.pad(position_ids.astype(jnp.int32),
                          ((0, 0), (0, SEQ_PAD - SEQ)))
    mesh = plsc.VectorSubcoreMesh(
        core_axis_name="c", subcore_axis_name="s",
        num_cores=NUM_CORES, num_subcores=NUM_SUBCORES)
    run = functools.partial(
        pl.kernel,
        out_type=jax.ShapeDtypeStruct((BATCH, SEQ, EMBED), jnp.float32),
        mesh=mesh,
        compiler_params=pltpu.CompilerParams(use_tc_tiling_on_sc=False),
        scratch_types=[
            pltpu.VMEM((SEQ_PAD,), jnp.int32),
            pltpu.VMEM((SEQ_PAD, EMBED), jnp.float32),
            pltpu.VMEM((SEQ_PAD,), jnp.int32),
            pltpu.VMEM((SEQ_PAD, EMBED), jnp.float32),
            pltpu.SemaphoreType.DMA,
        ],
    )(_impl)
    return run(ids_pad, pos_ids_pad, token_table, position_table)


# speedup vs baseline: 1.5080x; 1.5080x over previous
import functools

import jax
import jax.numpy as jnp
from jax import lax
from jax.experimental import pallas as pl
from jax.experimental.pallas import tpu as pltpu
from jax.experimental.pallas import tpu_sc as plsc

VOCAB = 49408
MAX_POS = 77
EMBED = 512
BATCH = 4096
SEQ = 77
SEQ_PAD = 80

NUM_CORES = 2
NUM_SUBCORES = 16
NUM_WORKERS = NUM_CORES * NUM_SUBCORES
BPW = BATCH // NUM_WORKERS
LANES = 16


def _impl(ids_hbm, pos_ids_hbm, tok_hbm, pos_hbm, out_hbm,
          idx_all, pos_idx, pos_rows, rows0, rows1,
          gsem0, gsem1, osem0, osem1):
    wid = lax.axis_index("s") * NUM_CORES + lax.axis_index("c")
    b0 = wid * BPW

    pltpu.sync_copy(ids_hbm.at[pl.ds(b0 * SEQ_PAD, BPW * SEQ_PAD)], idx_all)
    pltpu.sync_copy(pos_ids_hbm.at[0], pos_idx)
    pltpu.async_copy(pos_hbm.at[pos_idx.at[pl.ds(0, SEQ)]], pos_rows,
                     gsem0).wait()

    def idx_of(g):
        return idx_all.at[pl.ds(g * SEQ_PAD, SEQ)]

    def add_pos(rows):
        def add_row(r, c):
            for j in range(EMBED // LANES):
                sl = pl.ds(j * LANES, LANES)
                plsc.addupdate(rows.at[r, sl], pos_rows[r, sl])
            return c
        lax.fori_loop(0, SEQ, add_row, 0)

    pltpu.async_copy(tok_hbm.at[idx_of(0)], rows0, gsem0)
    pltpu.async_copy(tok_hbm.at[idx_of(1)], rows1, gsem1)

    def body(t, carry):
        g = 2 * t
        pltpu.make_async_copy(tok_hbm.at[idx_of(g)], rows0, gsem0).wait()
        add_pos(rows0)
        pltpu.async_copy(rows0, out_hbm.at[b0 + g], osem0)

        pltpu.make_async_copy(tok_hbm.at[idx_of(g + 1)], rows1, gsem1).wait()
        add_pos(rows1)
        pltpu.async_copy(rows1, out_hbm.at[b0 + g + 1], osem1)

        gn0 = jnp.minimum(g + 2, BPW - 1)
        gn1 = jnp.minimum(g + 3, BPW - 1)
        pltpu.make_async_copy(rows0, out_hbm.at[b0 + g], osem0).wait()
        pltpu.async_copy(tok_hbm.at[idx_of(gn0)], rows0, gsem0)
        pltpu.make_async_copy(rows1, out_hbm.at[b0 + g + 1], osem1).wait()
        pltpu.async_copy(tok_hbm.at[idx_of(gn1)], rows1, gsem1)
        return carry

    lax.fori_loop(0, BPW // 2, body, 0)

    pltpu.make_async_copy(tok_hbm.at[idx_of(BPW - 1)], rows0, gsem0).wait()
    pltpu.make_async_copy(tok_hbm.at[idx_of(BPW - 1)], rows1, gsem1).wait()


@jax.jit
def kernel(input_ids, position_ids, token_table, position_table):
    ids_pad = jnp.pad(input_ids.astype(jnp.int32),
                      ((0, 0), (0, SEQ_PAD - SEQ))).reshape(-1)
    pos_ids_pad = jnp.pad(position_ids.astype(jnp.int32),
                          ((0, 0), (0, SEQ_PAD - SEQ)))
    mesh = plsc.VectorSubcoreMesh(
        core_axis_name="c", subcore_axis_name="s",
        num_cores=NUM_CORES, num_subcores=NUM_SUBCORES)
    run = functools.partial(
        pl.kernel,
        out_type=jax.ShapeDtypeStruct((BATCH, SEQ, EMBED), jnp.float32),
        mesh=mesh,
        compiler_params=pltpu.CompilerParams(use_tc_tiling_on_sc=False),
        scratch_types=[
            pltpu.VMEM((BPW * SEQ_PAD,), jnp.int32),
            pltpu.VMEM((SEQ_PAD,), jnp.int32),
            pltpu.VMEM((SEQ, EMBED), jnp.float32),
            pltpu.VMEM((SEQ, EMBED), jnp.float32),
            pltpu.VMEM((SEQ, EMBED), jnp.float32),
            pltpu.SemaphoreType.DMA,
            pltpu.SemaphoreType.DMA,
            pltpu.SemaphoreType.DMA,
            pltpu.SemaphoreType.DMA,
        ],
    )(_impl)
    return run(ids_pad, pos_ids_pad, token_table, position_table)
